# baseline (speedup 1.0000x reference)
import jax
import jax.numpy as jnp
from jax import lax
from jax.experimental import pallas as pl
from jax.experimental.pallas import tpu as pltpu
from jax.experimental.pallas import tpu_sc as plsc

B = 4096
F = 26
D = 16
NUM_ROWS = 2600000
BF = B * F
NC = 2
NS = 16
NW = NC * NS
PER_W = BF // NW
CHUNK = 128
NCHUNK = PER_W // CHUNK

BLK = 512


def _sc_proj_body(idx_hbm, proj_hbm, pv_hbm, idx_v, pv_v, sem):
    wid = lax.axis_index("s") * NC + lax.axis_index("c")
    pltpu.sync_copy(idx_hbm.at[wid], idx_v)
    flat = proj_hbm.at[0]

    @pl.loop(0, NCHUNK)
    def _fire(j):
        pltpu.async_copy(flat.at[idx_v.at[j]], pv_v.at[j], sem)

    @pl.loop(0, NCHUNK)
    def _drain(j):
        pltpu.make_async_copy(flat.at[idx_v.at[j]], pv_v.at[j], sem).wait()

    pltpu.sync_copy(pv_v, pv_hbm.at[pl.ds(wid * NCHUNK, NCHUNK)])


def _sc_proj(idx3d, projT):
    mesh = plsc.VectorSubcoreMesh(core_axis_name="c", subcore_axis_name="s")
    k = pl.kernel(
        _sc_proj_body,
        out_type=jax.ShapeDtypeStruct((NW * NCHUNK, CHUNK), jnp.float32),
        mesh=mesh,
        compiler_params=pltpu.CompilerParams(use_tc_tiling_on_sc=False),
        scratch_types=[
            pltpu.VMEM((NCHUNK, CHUNK), jnp.int32),
            pltpu.VMEM((NCHUNK, CHUNK), jnp.float32),
            pltpu.SemaphoreType.DMA,
        ],
    )
    return k(idx3d, projT)


def _tc_body(vf_ref, pv_ref, w1_ref, b1_ref, w2_ref, b2_ref, w3_ref, b3_ref,
             w4_ref, scal_ref, out_ref):
    i = pl.program_id(0)
    bf = jnp.bfloat16
    w1b = w1_ref[...].astype(bf)
    s_t = jnp.zeros((D, BLK), jnp.float32)
    q = jnp.zeros((BLK,), jnp.float32)
    acc = jnp.zeros((BLK, 1024), jnp.float32)
    for f in range(F):
        vt_f = vf_ref[:, pl.ds(f * B + i * BLK, BLK)]
        s_t = s_t + vt_f
        q = q + jnp.sum(vt_f * vt_f, axis=0)
        acc = acc + lax.dot_general(
            vt_f.astype(bf), w1b[f * D:(f + 1) * D, :],
            (((0,), (0,)), ((), ())), preferred_element_type=jnp.float32)
    pv = pv_ref[...]

    fm_int = 0.5 * (jnp.sum(s_t * s_t, axis=0) - q)
    fm_lin = jnp.sum(pv, axis=1)
    fm_logit = (fm_lin + fm_int) * scal_ref[0] + scal_ref[1]

    h1 = jnp.maximum(acc + b1_ref[...], 0.0)
    h2 = jnp.maximum(
        jnp.dot(h1.astype(bf), w2_ref[...].astype(bf),
                preferred_element_type=jnp.float32) + b2_ref[...], 0.0)
    h3 = jnp.maximum(
        jnp.dot(h2.astype(bf), w3_ref[...].astype(bf),
                preferred_element_type=jnp.float32) + b3_ref[...], 0.0)
    mlp_logit = jnp.sum(h3 * w4_ref[...], axis=1) + scal_ref[2]

    out_ref[...] = jax.nn.sigmoid(fm_logit + mlp_logit)


def _tc_deepfm(vf, pv, w1t, b1, w2t, b2, w3t, b3, w4, scal):
    grid = (B // BLK,)
    full = lambda shape: pl.BlockSpec(shape, lambda i: (0, 0))
    return pl.pallas_call(
        _tc_body,
        grid=grid,
        in_specs=[
            pl.BlockSpec((D, B * F), lambda i: (0, 0)),
            pl.BlockSpec((BLK, F), lambda i: (i, 0)),
            full((F * D, 1024)),
            full((1, 1024)),
            full((1024, 512)),
            full((1, 512)),
            full((512, 256)),
            full((1, 256)),
            full((1, 256)),
            pl.BlockSpec(memory_space=pltpu.SMEM),
        ],
        out_specs=pl.BlockSpec((BLK,), lambda i: (i,)),
        out_shape=jax.ShapeDtypeStruct((B,), jnp.float32),
    )(vf, pv, w1t, b1, w2t, b2, w3t, b3, w4, scal)


def kernel(x, table, proj, fc_w, fc_b, W1, b1, W2, b2, W3, b3, W4, b4):
    idx3d = x.reshape(NW, NCHUNK, CHUNK)
    projT = proj.reshape(1, NUM_ROWS)
    pv = _sc_proj(idx3d, projT)
    pvb = pv.reshape(B, F)
    vf = jnp.take(table, x.T.reshape(-1), axis=0, mode="clip").T
    scal = jnp.concatenate([fc_w.reshape(-1), fc_b.reshape(-1),
                            b4.reshape(-1)])
    return _tc_deepfm(vf, pvb, W1.T, b1.reshape(1, -1), W2.T,
                      b2.reshape(1, -1), W3.T, b3.reshape(1, -1),
                      W4.reshape(1, -1), scal)


# speedup vs baseline: 1.1144x; 1.1144x over previous
import jax
import jax.numpy as jnp
from jax import lax
from jax.experimental import pallas as pl
from jax.experimental.pallas import tpu as pltpu
from jax.experimental.pallas import tpu_sc as plsc

B = 4096
F = 26
D = 16
NUM_ROWS = 2600000
BF = B * F
NC = 2
NS = 16
NW = NC * NS
PER_W = BF // NW
CHUNK = 128
NCHUNK = PER_W // CHUNK

BLK = 1024


def _sc_proj_body(idx_hbm, proj_hbm, pv_hbm, idx_v, pv_v, sem):
    wid = lax.axis_index("s") * NC + lax.axis_index("c")
    pltpu.sync_copy(idx_hbm.at[wid], idx_v)
    flat = proj_hbm.at[0]

    @pl.loop(0, NCHUNK)
    def _fire(j):
        pltpu.async_copy(flat.at[idx_v.at[j]], pv_v.at[j], sem)

    @pl.loop(0, NCHUNK)
    def _drain(j):
        pltpu.make_async_copy(flat.at[idx_v.at[j]], pv_v.at[j], sem).wait()

    pltpu.sync_copy(pv_v, pv_hbm.at[pl.ds(wid * NCHUNK, NCHUNK)])


def _sc_proj(idx3d, projT):
    mesh = plsc.VectorSubcoreMesh(core_axis_name="c", subcore_axis_name="s")
    k = pl.kernel(
        _sc_proj_body,
        out_type=jax.ShapeDtypeStruct((NW * NCHUNK, CHUNK), jnp.float32),
        mesh=mesh,
        compiler_params=pltpu.CompilerParams(use_tc_tiling_on_sc=False),
        scratch_types=[
            pltpu.VMEM((NCHUNK, CHUNK), jnp.int32),
            pltpu.VMEM((NCHUNK, CHUNK), jnp.float32),
            pltpu.SemaphoreType.DMA,
        ],
    )
    return k(idx3d, projT)


def _tc_body(vf_ref, pv_ref, w1_ref, b1_ref, w2_ref, b2_ref, w3_ref, b3_ref,
             w4_ref, scal_ref, out_ref):
    i = pl.program_id(0)
    parts = []
    for f in range(F):
        sl = vf_ref[:, pl.ds(f * B + i * BLK, BLK)]
        parts.append(jnp.transpose(sl))
    h = jnp.concatenate(parts, axis=1)
    pv = pv_ref[...]

    row = lax.broadcasted_iota(jnp.int32, (F * D, D), 0)
    col = lax.broadcasted_iota(jnp.int32, (F * D, D), 1)
    m = (row % D == col).astype(jnp.float32)
    s = jnp.dot(h, m, preferred_element_type=jnp.float32)
    fm_int = 0.5 * (jnp.sum(s * s, axis=1) - jnp.sum(h * h, axis=1))
    fm_lin = jnp.sum(pv, axis=1)
    fm_logit = (fm_lin + fm_int) * scal_ref[0] + scal_ref[1]

    bf = jnp.bfloat16
    h1 = jnp.maximum(
        jnp.dot(h.astype(bf), w1_ref[...].astype(bf),
                preferred_element_type=jnp.float32) + b1_ref[...], 0.0)
    h2 = jnp.maximum(
        jnp.dot(h1.astype(bf), w2_ref[...].astype(bf),
                preferred_element_type=jnp.float32) + b2_ref[...], 0.0)
    h3 = jnp.maximum(
        jnp.dot(h2.astype(bf), w3_ref[...].astype(bf),
                preferred_element_type=jnp.float32) + b3_ref[...], 0.0)
    mlp_logit = jnp.sum(h3 * w4_ref[...], axis=1) + scal_ref[2]

    out_ref[...] = jax.nn.sigmoid(fm_logit + mlp_logit)


def _tc_deepfm(vf, pv, w1t, b1, w2t, b2, w3t, b3, w4, scal):
    grid = (B // BLK,)
    full = lambda shape: pl.BlockSpec(shape, lambda i: (0, 0))
    return pl.pallas_call(
        _tc_body,
        grid=grid,
        in_specs=[
            pl.BlockSpec((D, B * F), lambda i: (0, 0)),
            pl.BlockSpec((BLK, F), lambda i: (i, 0)),
            full((F * D, 1024)),
            full((1, 1024)),
            full((1024, 512)),
            full((1, 512)),
            full((512, 256)),
            full((1, 256)),
            full((1, 256)),
            pl.BlockSpec(memory_space=pltpu.SMEM),
        ],
        out_specs=pl.BlockSpec((BLK,), lambda i: (i,)),
        out_shape=jax.ShapeDtypeStruct((B,), jnp.float32),
    )(vf, pv, w1t, b1, w2t, b2, w3t, b3, w4, scal)


def kernel(x, table, proj, fc_w, fc_b, W1, b1, W2, b2, W3, b3, W4, b4):
    idx3d = x.reshape(NW, NCHUNK, CHUNK)
    projT = proj.reshape(1, NUM_ROWS)
    pv = _sc_proj(idx3d, projT)
    pvb = pv.reshape(B, F)
    vf = jnp.take(table, x.T.reshape(-1), axis=0, mode="clip").T
    scal = jnp.concatenate([fc_w.reshape(-1), fc_b.reshape(-1),
                            b4.reshape(-1)])
    return _tc_deepfm(vf, pvb, W1.T, b1.reshape(1, -1), W2.T,
                      b2.reshape(1, -1), W3.T, b3.reshape(1, -1),
                      W4.reshape(1, -1), scal)


# speedup vs baseline: 1.1176x; 1.0029x over previous
import jax
import jax.numpy as jnp
from jax import lax
from jax.experimental import pallas as pl
from jax.experimental.pallas import tpu as pltpu
from jax.experimental.pallas import tpu_sc as plsc

B = 4096
F = 26
D = 16
NUM_ROWS = 2600000
BF = B * F
NC = 2
NS = 16
NW = NC * NS
PER_W = BF // NW
CHUNK = 128
NCHUNK = PER_W // CHUNK

BLK = 2048


def _sc_proj_body(idx_hbm, proj_hbm, pv_hbm, idx_v, pv_v, sem):
    wid = lax.axis_index("s") * NC + lax.axis_index("c")
    pltpu.sync_copy(idx_hbm.at[wid], idx_v)
    flat = proj_hbm.at[0]

    @pl.loop(0, NCHUNK)
    def _fire(j):
        pltpu.async_copy(flat.at[idx_v.at[j]], pv_v.at[j], sem)

    @pl.loop(0, NCHUNK)
    def _drain(j):
        pltpu.make_async_copy(flat.at[idx_v.at[j]], pv_v.at[j], sem).wait()

    pltpu.sync_copy(pv_v, pv_hbm.at[pl.ds(wid * NCHUNK, NCHUNK)])


def _sc_proj(idx3d, projT):
    mesh = plsc.VectorSubcoreMesh(core_axis_name="c", subcore_axis_name="s")
    k = pl.kernel(
        _sc_proj_body,
        out_type=jax.ShapeDtypeStruct((NW * NCHUNK, CHUNK), jnp.float32),
        mesh=mesh,
        compiler_params=pltpu.CompilerParams(use_tc_tiling_on_sc=False),
        scratch_types=[
            pltpu.VMEM((NCHUNK, CHUNK), jnp.int32),
            pltpu.VMEM((NCHUNK, CHUNK), jnp.float32),
            pltpu.SemaphoreType.DMA,
        ],
    )
    return k(idx3d, projT)


def _tc_body(vf_ref, pv_ref, w1_ref, b1_ref, w2_ref, b2_ref, w3_ref, b3_ref,
             w4_ref, scal_ref, out_ref):
    i = pl.program_id(0)
    parts = []
    for f in range(F):
        sl = vf_ref[:, pl.ds(f * B + i * BLK, BLK)]
        parts.append(jnp.transpose(sl))
    h = jnp.concatenate(parts, axis=1)
    pv = pv_ref[...]

    row = lax.broadcasted_iota(jnp.int32, (F * D, D), 0)
    col = lax.broadcasted_iota(jnp.int32, (F * D, D), 1)
    m = (row % D == col).astype(jnp.float32)
    s = jnp.dot(h, m, preferred_element_type=jnp.float32)
    fm_int = 0.5 * (jnp.sum(s * s, axis=1) - jnp.sum(h * h, axis=1))
    fm_lin = jnp.sum(pv, axis=1)
    fm_logit = (fm_lin + fm_int) * scal_ref[0] + scal_ref[1]

    bf = jnp.bfloat16
    h1 = jnp.maximum(
        jnp.dot(h.astype(bf), w1_ref[...].astype(bf),
                preferred_element_type=jnp.float32) + b1_ref[...], 0.0)
    h2 = jnp.maximum(
        jnp.dot(h1.astype(bf), w2_ref[...].astype(bf),
                preferred_element_type=jnp.float32) + b2_ref[...], 0.0)
    h3 = jnp.maximum(
        jnp.dot(h2.astype(bf), w3_ref[...].astype(bf),
                preferred_element_type=jnp.float32) + b3_ref[...], 0.0)
    mlp_logit = jnp.sum(h3 * w4_ref[...], axis=1) + scal_ref[2]

    out_ref[...] = jax.nn.sigmoid(fm_logit + mlp_logit)


def _tc_deepfm(vf, pv, w1t, b1, w2t, b2, w3t, b3, w4, scal):
    grid = (B // BLK,)
    full = lambda shape: pl.BlockSpec(shape, lambda i: (0, 0))
    return pl.pallas_call(
        _tc_body,
        grid=grid,
        in_specs=[
            pl.BlockSpec((D, B * F), lambda i: (0, 0)),
            pl.BlockSpec((BLK, F), lambda i: (i, 0)),
            full((F * D, 1024)),
            full((1, 1024)),
            full((1024, 512)),
            full((1, 512)),
            full((512, 256)),
            full((1, 256)),
            full((1, 256)),
            pl.BlockSpec(memory_space=pltpu.SMEM),
        ],
        out_specs=pl.BlockSpec((BLK,), lambda i: (i,)),
        out_shape=jax.ShapeDtypeStruct((B,), jnp.float32),
    )(vf, pv, w1t, b1, w2t, b2, w3t, b3, w4, scal)


def kernel(x, table, proj, fc_w, fc_b, W1, b1, W2, b2, W3, b3, W4, b4):
    idx3d = x.reshape(NW, NCHUNK, CHUNK)
    projT = proj.reshape(1, NUM_ROWS)
    pv = _sc_proj(idx3d, projT)
    pvb = pv.reshape(B, F)
    vf = jnp.take(table, x.T.reshape(-1), axis=0, mode="clip").T
    scal = jnp.concatenate([fc_w.reshape(-1), fc_b.reshape(-1),
                            b4.reshape(-1)])
    return _tc_deepfm(vf, pvb, W1.T, b1.reshape(1, -1), W2.T,
                      b2.reshape(1, -1), W3.T, b3.reshape(1, -1),
                      W4.reshape(1, -1), scal)
